# Initial kernel scaffold; baseline (speedup 1.0000x reference)
import jax
import jax.numpy as jnp
from jax.experimental import pallas as pl


def kernel(x, edge_index, W1, b1, W2, b2):
    raise NotImplementedError("write your pallas kernel here")


# speedup vs baseline: 35.2363x; 35.2363x over previous
import functools

import jax
import jax.numpy as jnp
from jax import lax
from jax.experimental import pallas as pl
from jax.experimental.pallas import tpu as pltpu
from jax.experimental.pallas import tpu_sc as plsc

N = 10000
E = 320000
D = 128
D_OUT = 64
K = 1000
KP = 1024
NP = 10240
DROW = 80
SENT = 2047
ET = 321024
PT = 10032
PD = 20000
CAP = 4096
CROWS = 64
DCH = 2000
ECH = 1264
ECH_SIZES = (1264,) * 7 + (1184,)


def _consts():
    perm = jax.random.permutation(jax.random.key(42), N)
    sel = perm[:K].astype(jnp.int32)
    pos = jnp.full((NP,), SENT, jnp.int32)
    pos = pos.at[sel].set(jnp.arange(K, dtype=jnp.int32))
    sel_pad = jnp.zeros((KP,), jnp.int32).at[:K].set(sel)
    return sel, pos, sel_pad


def _rsqrt_newton(d):
    i = plsc.bitcast(d, jnp.int32)
    i = jnp.int32(0x5F3759DF) - (i >> 1)
    y = plsc.bitcast(i, jnp.float32)
    for _ in range(3):
        y = y * (1.5 - 0.5 * d * y * y)
    return y


def _sc_body(src_hbm, dst_hbm, dstdeg_hbm, pos_hbm, sel_hbm, x_hbm,
             acc_out, dsel_out, xs_out,
             pos_v, deg_v, ebuf_s, ebuf_d, ec, rows,
             idxq, iq1, iq2, iq3, deg_sh, acc_sh, sem):
    c = lax.axis_index("c")
    s = lax.axis_index("s")
    wid = c * 16 + s
    iota16 = lax.iota(jnp.int32, 16)
    zf16 = jnp.zeros((16,), jnp.float32)
    zi16 = jnp.zeros((16,), jnp.int32)

    def _zero_deg(i, _):
        for q in range(8):
            deg_v[i, pl.ds(q * 16, 16)] = zf16
        return 0
    lax.fori_loop(0, DROW, _zero_deg, 0)

    for k in range(5):
        idxq[0, pl.ds(k * 16, 16)] = iota16 + k * 16

    def _zero_rows(r, _):
        for q in range(8):
            rows[r, pl.ds(q * 16, 16)] = zf16
        return 0
    lax.fori_loop(0, 64, _zero_rows, 0)

    @pl.when(s == 0)
    def _():
        pltpu.sync_copy(deg_v, deg_sh)

    pltpu.sync_copy(rows, acc_sh.at[pl.ds(s * 64, 64)])

    pltpu.sync_copy(pos_hbm, pos_v)

    for j in range(PD // DCH):
        base = s * PD + j * DCH
        pltpu.sync_copy(dstdeg_hbm.at[pl.ds(base, DCH)],
                        ebuf_d.at[pl.ds(0, DCH)])

        def _deg(i, _):
            d16 = ebuf_d[pl.ds(i * 16, 16)]
            cnt, last = plsc.scan_count(d16)
            plsc.addupdate_scatter(
                deg_v, [d16 >> 7, d16 & 127],
                cnt.astype(jnp.float32), mask=last)
            return 0
        lax.fori_loop(0, DCH // 16, _deg, 0)

    plsc.subcore_barrier()

    pltpu.sync_copy(deg_v, deg_sh.at[idxq.at[0]], add=True)

    plsc.subcore_barrier()

    pltpu.sync_copy(deg_sh, deg_v)

    def _newton(i, _):
        for q in range(8):
            d = deg_v[i, pl.ds(q * 16, 16)] + 1.0
            deg_v[i, pl.ds(q * 16, 16)] = _rsqrt_newton(d)
        return 0
    lax.fori_loop(0, DROW, _newton, 0)

    def _xs_chunk(jj, _):
        base = s * 625 + jnp.minimum(jj * 64, 561)
        for q in range(4):
            iq1[0, pl.ds(q * 16, 16)] = base + q * 16 + iota16
        pltpu.async_copy(x_hbm.at[iq1.at[0]], rows, sem).wait()

        def _scale_row(r, _2):
            g = base + r
            w = plsc.load_gather(
                deg_v, [jnp.full((16,), g >> 7, jnp.int32),
                        jnp.full((16,), g & 127, jnp.int32)])
            for q in range(8):
                rows[r, pl.ds(q * 16, 16)] = rows[r, pl.ds(q * 16, 16)] * w
            return 0
        lax.fori_loop(0, 64, _scale_row, 0)
        pltpu.async_copy(rows, xs_out.at[c].at[iq1.at[0]], sem).wait()
        return 0
    lax.fori_loop(0, 10, _xs_chunk, 0)

    cnt16 = zi16
    for j, csz in enumerate(ECH_SIZES):
        ebase = wid * PT + j * ECH
        pltpu.sync_copy(src_hbm.at[pl.ds(ebase, csz)],
                        ebuf_s.at[pl.ds(0, csz)])
        pltpu.sync_copy(dst_hbm.at[pl.ds(ebase, csz)],
                        ebuf_d.at[pl.ds(0, csz)])

        def _compact(i, cnt16):
            s16 = ebuf_s[pl.ds(i * 16, 16)]
            d16 = ebuf_d[pl.ds(i * 16, 16)]
            p16 = plsc.load_gather(pos_v, [d16])
            m = p16 < K
            pre = jnp.cumsum(m.astype(jnp.int32))
            offs = jnp.minimum(cnt16 + pre - 1, CAP - 1)
            plsc.store_scatter(ec, [offs >> 6, offs & 63],
                               (p16 << 14) | s16, mask=m)
            return cnt16 + pre[15]
        cnt16 = lax.fori_loop(0, csz // 16, _compact, cnt16)
    cnt = jnp.minimum(cnt16[0], CAP)
    nch = (cnt + 63) >> 6

    def _pad(t, _):
        idx = t * 16 + iota16
        m = idx >= cnt
        plsc.store_scatter(ec, [idx >> 6, idx & 63],
                           jnp.full((16,), 1023 << 14, jnp.int32), mask=m)
        return 0
    lax.fori_loop(cnt >> 4, nch * 4, _pad, 0)

    plsc.subcore_barrier()

    def _chunk(g, _):
        for q in range(4):
            v = ec[g, pl.ds(q * 16, 16)]
            iq2[0, pl.ds(q * 16, 16)] = v & 16383
            iq3[0, pl.ds(q * 16, 16)] = v >> 14
        pltpu.async_copy(xs_out.at[c].at[iq2.at[0]], rows, sem).wait()
        pltpu.sync_copy(rows, acc_sh.at[iq3.at[0]], add=True)
        return 0
    lax.fori_loop(0, nch, _chunk, 0)

    plsc.subcore_barrier()

    for q in range(4):
        iq1[0, pl.ds(q * 16, 16)] = s * 64 + q * 16 + iota16
    pltpu.sync_copy(acc_sh.at[pl.ds(s * 64, 64)], rows)
    pltpu.async_copy(rows, acc_out.at[c].at[iq1.at[0]], sem).wait()

    @pl.when((c == 0) & (s == 0))
    def _():
        pltpu.sync_copy(sel_hbm, ebuf_s.at[pl.ds(0, KP)])

        def _dsel(k, _):
            i16 = ebuf_s[pl.ds(k * 16, 16)]
            v16 = plsc.load_gather(deg_v, [i16 >> 7, i16 & 127])
            ebuf_d[pl.ds(k * 16, 16)] = plsc.bitcast(v16, jnp.int32)
            return 0
        lax.fori_loop(0, KP // 16, _dsel, 0)
        pltpu.sync_copy(ebuf_d.at[pl.ds(0, KP)], dsel_out)


_sc_kernel = functools.partial(
    pl.kernel,
    out_type=(
        jax.ShapeDtypeStruct((2, KP, D), jnp.float32),
        jax.ShapeDtypeStruct((KP,), jnp.int32),
        jax.ShapeDtypeStruct((2, N, D), jnp.float32),
    ),
    mesh=plsc.VectorSubcoreMesh(core_axis_name="c", subcore_axis_name="s"),
    compiler_params=pltpu.CompilerParams(needs_layout_passes=False),
    scratch_types=(
        pltpu.VMEM((NP,), jnp.int32),
        pltpu.VMEM((DROW, 128), jnp.float32),
        pltpu.VMEM((ECH,), jnp.int32),
        pltpu.VMEM((DCH,), jnp.int32),
        pltpu.VMEM((CROWS, 64), jnp.int32),
        pltpu.VMEM((64, D), jnp.float32),
        pltpu.VMEM((1, 80), jnp.int32),
        pltpu.VMEM((1, 64), jnp.int32),
        pltpu.VMEM((1, 64), jnp.int32),
        pltpu.VMEM((1, 64), jnp.int32),
        pltpu.VMEM_SHARED((DROW, 128), jnp.float32),
        pltpu.VMEM_SHARED((KP, D), jnp.float32),
        pltpu.SemaphoreType.DMA,
    ),
)(_sc_body)


def _tc_body(acc_ref, dsel_ref, w1_ref, b1_ref, w2_ref, b2_ref, out_ref):
    a = (acc_ref[0] + acc_ref[1]) * dsel_ref[...]
    z = jnp.dot(a, w1_ref[...], precision=jax.lax.Precision.HIGHEST)
    z = jnp.maximum(z + b1_ref[...], 0.0)
    o = jnp.dot(z, w2_ref[...], precision=jax.lax.Precision.HIGHEST)
    o = o + b2_ref[...]
    m = jnp.max(o, axis=1, keepdims=True)
    lse = jnp.log(jnp.sum(jnp.exp(o - m), axis=1, keepdims=True)) + m
    out_ref[...] = (o - lse)[:K]


def kernel(x, edge_index, W1, b1, W2, b2):
    sel, pos1d, sel_pad = _consts()
    src_ext = jnp.concatenate(
        [edge_index[0], sel, jnp.zeros((24,), jnp.int32)])
    dst_ext = jnp.concatenate(
        [edge_index[1], sel, jnp.full((24,), N, jnp.int32)])
    acc, dsel_bits, _ = _sc_kernel(
        src_ext, dst_ext, edge_index[1], pos1d, sel_pad, x)
    dsel = lax.bitcast_convert_type(dsel_bits, jnp.float32)
    out = pl.pallas_call(
        _tc_body,
        out_shape=jax.ShapeDtypeStruct((K, D_OUT), jnp.float32),
    )(acc, dsel.reshape(KP, 1), W1, b1.reshape(1, D), W2,
      b2.reshape(1, D_OUT))
    return out


# speedup vs baseline: 36.1829x; 1.0269x over previous
import functools

import jax
import jax.numpy as jnp
from jax import lax
from jax.experimental import pallas as pl
from jax.experimental.pallas import tpu as pltpu
from jax.experimental.pallas import tpu_sc as plsc

N = 10000
E = 320000
D = 128
D_OUT = 64
K = 1000
KP = 1024
NP = 10240
DROW = 80
SENT = 2047
ET = 321024
PT = 10032
PD = 20000
CAP = 4096
CROWS = 64
DCH = 2000
ECH = 1264
ECH_SIZES = (1264,) * 7 + (1184,)


def _consts():
    perm = jax.random.permutation(jax.random.key(42), N)
    sel = perm[:K].astype(jnp.int32)
    pos = jnp.full((NP,), SENT, jnp.int32)
    pos = pos.at[sel].set(jnp.arange(K, dtype=jnp.int32))
    sel_pad = jnp.zeros((KP,), jnp.int32).at[:K].set(sel)
    return sel, pos, sel_pad


def _rsqrt_newton(d):
    i = plsc.bitcast(d, jnp.int32)
    i = jnp.int32(0x5F3759DF) - (i >> 1)
    y = plsc.bitcast(i, jnp.float32)
    for _ in range(3):
        y = y * (1.5 - 0.5 * d * y * y)
    return y


def _sc_body(src_hbm, dst_hbm, dstdeg_hbm, pos_hbm, sel_hbm, x_hbm,
             acc_out, dsel_out, xs_out,
             pos_v, deg_v, ebuf_s, ebuf_d, ec, rows,
             idxq, iq1, iq2, iq3, deg_sh, acc_sh, sem, gsem, ssem):
    c = lax.axis_index("c")
    s = lax.axis_index("s")
    wid = c * 16 + s
    iota16 = lax.iota(jnp.int32, 16)
    zf16 = jnp.zeros((16,), jnp.float32)
    zi16 = jnp.zeros((16,), jnp.int32)

    def _zero_deg(i, _):
        for q in range(8):
            deg_v[i, pl.ds(q * 16, 16)] = zf16
        return 0
    lax.fori_loop(0, DROW, _zero_deg, 0)

    for k in range(5):
        idxq[0, pl.ds(k * 16, 16)] = iota16 + k * 16

    def _zero_rows(r, _):
        for b in range(2):
            for q in range(8):
                rows[b, r, pl.ds(q * 16, 16)] = zf16
        return 0
    lax.fori_loop(0, 32, _zero_rows, 0)

    @pl.when(s == 0)
    def _():
        pltpu.sync_copy(deg_v, deg_sh)

    pltpu.sync_copy(rows.at[0], acc_sh.at[pl.ds(s * 64, 32)])
    pltpu.sync_copy(rows.at[1], acc_sh.at[pl.ds(s * 64 + 32, 32)])

    pltpu.sync_copy(pos_hbm, pos_v)

    for j in range(PD // DCH):
        base = s * PD + j * DCH
        pltpu.sync_copy(dstdeg_hbm.at[pl.ds(base, DCH)],
                        ebuf_d.at[pl.ds(0, DCH)])

        def _deg(i, _):
            d16 = ebuf_d[pl.ds(i * 16, 16)]
            cnt, last = plsc.scan_count(d16)
            plsc.addupdate_scatter(
                deg_v, [d16 >> 7, d16 & 127],
                cnt.astype(jnp.float32), mask=last)
            return 0
        lax.fori_loop(0, DCH // 16, _deg, 0)

    plsc.subcore_barrier()

    pltpu.sync_copy(deg_v, deg_sh.at[idxq.at[0]], add=True)

    plsc.subcore_barrier()

    pltpu.sync_copy(deg_sh, deg_v)

    def _newton(i, _):
        for q in range(8):
            d = deg_v[i, pl.ds(q * 16, 16)] + 1.0
            deg_v[i, pl.ds(q * 16, 16)] = _rsqrt_newton(d)
        return 0
    lax.fori_loop(0, DROW, _newton, 0)

    def _xs_chunk(jj, _):
        base = s * 625 + jnp.minimum(jj * 32, 593)
        for q in range(2):
            iq1[0, pl.ds(q * 16, 16)] = base + q * 16 + iota16
        pltpu.async_copy(x_hbm.at[iq1.at[0]], rows.at[0], sem).wait()

        def _scale_row(r, _2):
            g = base + r
            w = plsc.load_gather(
                deg_v, [jnp.full((16,), g >> 7, jnp.int32),
                        jnp.full((16,), g & 127, jnp.int32)])
            for q in range(8):
                rows[0, r, pl.ds(q * 16, 16)] = (
                    rows[0, r, pl.ds(q * 16, 16)] * w)
            return 0
        lax.fori_loop(0, 32, _scale_row, 0)
        pltpu.async_copy(rows.at[0], xs_out.at[c].at[iq1.at[0]], sem).wait()
        return 0
    lax.fori_loop(0, 20, _xs_chunk, 0)

    cnt16 = zi16
    for j, csz in enumerate(ECH_SIZES):
        ebase = wid * PT + j * ECH
        pltpu.sync_copy(src_hbm.at[pl.ds(ebase, csz)],
                        ebuf_s.at[pl.ds(0, csz)])
        pltpu.sync_copy(dst_hbm.at[pl.ds(ebase, csz)],
                        ebuf_d.at[pl.ds(0, csz)])

        def _compact(i, cnt16):
            s16 = ebuf_s[pl.ds(i * 16, 16)]
            d16 = ebuf_d[pl.ds(i * 16, 16)]
            p16 = plsc.load_gather(pos_v, [d16])
            m = p16 < K
            pre = jnp.cumsum(m.astype(jnp.int32))
            offs = jnp.minimum(cnt16 + pre - 1, CAP - 1)
            plsc.store_scatter(ec, [offs >> 6, offs & 63],
                               (p16 << 14) | s16, mask=m)
            return cnt16 + pre[15]
        cnt16 = lax.fori_loop(0, csz // 16, _compact, cnt16)
    cnt = jnp.minimum(cnt16[0], CAP)
    nch = (cnt + 31) >> 5

    def _pad(t, _):
        idx = t * 16 + iota16
        m = idx >= cnt
        plsc.store_scatter(ec, [idx >> 6, idx & 63],
                           jnp.full((16,), 1023 << 14, jnp.int32), mask=m)
        return 0
    lax.fori_loop(cnt >> 4, nch * 2, _pad, 0)

    plsc.subcore_barrier()

    xs_c = xs_out.at[c]

    def _dec(g, b):
        for q in range(2):
            v = ec[g >> 1, pl.ds((g & 1) * 32 + q * 16, 16)]
            iq2[b, pl.ds(q * 16, 16)] = v & 16383
            iq3[b, pl.ds(q * 16, 16)] = v >> 14

    def _gather_start(b):
        pltpu.async_copy(xs_c.at[iq2.at[b]], rows.at[b], gsem.at[b])

    def _gather_wait(b):
        pltpu.make_async_copy(xs_c.at[iq2.at[b]], rows.at[b],
                              gsem.at[b]).wait()

    def _scat_start(b):
        pltpu.async_copy(rows.at[b], acc_sh.at[iq3.at[b]], ssem.at[b],
                         add=True)

    def _scat_wait(b):
        pltpu.make_async_copy(rows.at[b], acc_sh.at[iq3.at[b]],
                              ssem.at[b]).wait()

    @pl.when(nch > 0)
    def _():
        _dec(0, 0)
        _gather_start(0)

    def _chunk(g, _):
        b = g & 1
        nb = 1 - b

        @pl.when(g + 1 < nch)
        def _():
            @pl.when(g >= 1)
            def _():
                _scat_wait(nb)
            _dec(g + 1, nb)
            _gather_start(nb)
        _gather_wait(b)
        _scat_start(b)
        return 0
    lax.fori_loop(0, nch, _chunk, 0)

    @pl.when(nch >= 2)
    def _():
        _scat_wait(nch & 1)
    @pl.when(nch >= 1)
    def _():
        _scat_wait((nch - 1) & 1)

    plsc.subcore_barrier()

    for h in range(2):
        for q in range(2):
            iq1[h, pl.ds(q * 16, 16)] = s * 64 + h * 32 + q * 16 + iota16
        pltpu.sync_copy(acc_sh.at[pl.ds(s * 64 + h * 32, 32)], rows.at[h])
        pltpu.async_copy(rows.at[h], acc_out.at[c].at[iq1.at[h]],
                         sem).wait()

    @pl.when((c == 0) & (s == 0))
    def _():
        pltpu.sync_copy(sel_hbm, ebuf_s.at[pl.ds(0, KP)])

        def _dsel(k, _):
            i16 = ebuf_s[pl.ds(k * 16, 16)]
            v16 = plsc.load_gather(deg_v, [i16 >> 7, i16 & 127])
            ebuf_d[pl.ds(k * 16, 16)] = plsc.bitcast(v16, jnp.int32)
            return 0
        lax.fori_loop(0, KP // 16, _dsel, 0)
        pltpu.sync_copy(ebuf_d.at[pl.ds(0, KP)], dsel_out)


_sc_kernel = functools.partial(
    pl.kernel,
    out_type=(
        jax.ShapeDtypeStruct((2, KP, D), jnp.float32),
        jax.ShapeDtypeStruct((KP,), jnp.int32),
        jax.ShapeDtypeStruct((2, N, D), jnp.float32),
    ),
    mesh=plsc.VectorSubcoreMesh(core_axis_name="c", subcore_axis_name="s"),
    compiler_params=pltpu.CompilerParams(needs_layout_passes=False),
    scratch_types=(
        pltpu.VMEM((NP,), jnp.int32),
        pltpu.VMEM((DROW, 128), jnp.float32),
        pltpu.VMEM((ECH,), jnp.int32),
        pltpu.VMEM((DCH,), jnp.int32),
        pltpu.VMEM((CROWS, 64), jnp.int32),
        pltpu.VMEM((2, 32, D), jnp.float32),
        pltpu.VMEM((1, 80), jnp.int32),
        pltpu.VMEM((2, 32), jnp.int32),
        pltpu.VMEM((2, 32), jnp.int32),
        pltpu.VMEM((2, 32), jnp.int32),
        pltpu.VMEM_SHARED((DROW, 128), jnp.float32),
        pltpu.VMEM_SHARED((KP, D), jnp.float32),
        pltpu.SemaphoreType.DMA,
        pltpu.SemaphoreType.DMA((2,)),
        pltpu.SemaphoreType.DMA((2,)),
    ),
)(_sc_body)


def _tc_body(acc_ref, dsel_ref, w1_ref, b1_ref, w2_ref, b2_ref, out_ref):
    a = (acc_ref[0] + acc_ref[1]) * dsel_ref[...]
    z = jnp.dot(a, w1_ref[...], precision=jax.lax.Precision.HIGHEST)
    z = jnp.maximum(z + b1_ref[...], 0.0)
    o = jnp.dot(z, w2_ref[...], precision=jax.lax.Precision.HIGHEST)
    o = o + b2_ref[...]
    m = jnp.max(o, axis=1, keepdims=True)
    lse = jnp.log(jnp.sum(jnp.exp(o - m), axis=1, keepdims=True)) + m
    out_ref[...] = (o - lse)[:K]


def kernel(x, edge_index, W1, b1, W2, b2):
    sel, pos1d, sel_pad = _consts()
    src_ext = jnp.concatenate(
        [edge_index[0], sel, jnp.zeros((24,), jnp.int32)])
    dst_ext = jnp.concatenate(
        [edge_index[1], sel, jnp.full((24,), N, jnp.int32)])
    acc, dsel_bits, _ = _sc_kernel(
        src_ext, dst_ext, edge_index[1], pos1d, sel_pad, x)
    dsel = lax.bitcast_convert_type(dsel_bits, jnp.float32)
    out = pl.pallas_call(
        _tc_body,
        out_shape=jax.ShapeDtypeStruct((K, D_OUT), jnp.float32),
    )(acc, dsel.reshape(KP, 1), W1, b1.reshape(1, D), W2,
      b2.reshape(1, D_OUT))
    return out


# speedup vs baseline: 36.2245x; 1.0011x over previous
import functools

import jax
import jax.numpy as jnp
from jax import lax
from jax.experimental import pallas as pl
from jax.experimental.pallas import tpu as pltpu
from jax.experimental.pallas import tpu_sc as plsc

N = 10000
E = 320000
D = 128
D_OUT = 64
K = 1000
KP = 1024
NP = 10240
DROW = 80
SENT = 2047
ET = 321024
PT = 10032
PD = 20000
CAP = 4096
CROWS = 64
DCH = 2000
ECH = 1264
ECH_SIZES = (1264,) * 7 + (1184,)


def _consts():
    perm = jax.random.permutation(jax.random.key(42), N)
    sel = perm[:K].astype(jnp.int32)
    pos = jnp.full((NP,), SENT, jnp.int32)
    pos = pos.at[sel].set(jnp.arange(K, dtype=jnp.int32))
    sel_pad = jnp.zeros((KP,), jnp.int32).at[:K].set(sel)
    return sel, pos, sel_pad


def _rsqrt_newton(d):
    i = plsc.bitcast(d, jnp.int32)
    i = jnp.int32(0x5F3759DF) - (i >> 1)
    y = plsc.bitcast(i, jnp.float32)
    for _ in range(3):
        y = y * (1.5 - 0.5 * d * y * y)
    return y


def _sc_body(src_hbm, dst_hbm, dstdeg_hbm, pos_hbm, sel_hbm, x_hbm,
             acc_out, dsel_out, xs_out,
             pos_v, deg_v, ebuf_s, ebuf_d, ec, rows,
             idxq, iq1, iq2, iq3, deg_sh, acc_sh, sem, gsem, ssem):
    c = lax.axis_index("c")
    s = lax.axis_index("s")
    wid = c * 16 + s
    iota16 = lax.iota(jnp.int32, 16)
    zf16 = jnp.zeros((16,), jnp.float32)
    zi16 = jnp.zeros((16,), jnp.int32)

    def _zero_deg(i, _):
        for q in range(8):
            deg_v[i, pl.ds(q * 16, 16)] = zf16
        return 0
    lax.fori_loop(0, DROW, _zero_deg, 0)

    for k in range(5):
        idxq[0, pl.ds(k * 16, 16)] = iota16 + k * 16

    def _zero_rows(r, _):
        for b in range(2):
            for q in range(8):
                rows[b, r, pl.ds(q * 16, 16)] = zf16
        return 0
    lax.fori_loop(0, 32, _zero_rows, 0)

    @pl.when(s == 0)
    def _():
        pltpu.sync_copy(deg_v, deg_sh)

    pltpu.sync_copy(rows.at[0], acc_sh.at[pl.ds(s * 64, 32)])
    pltpu.sync_copy(rows.at[1], acc_sh.at[pl.ds(s * 64 + 32, 32)])

    pltpu.sync_copy(pos_hbm, pos_v)

    for j in range(PD // DCH):
        base = s * PD + j * DCH
        pltpu.sync_copy(dstdeg_hbm.at[pl.ds(base, DCH)],
                        ebuf_d.at[pl.ds(0, DCH)])

        def _deg(i, _):
            d16 = ebuf_d[pl.ds(i * 16, 16)]
            cnt, last = plsc.scan_count(d16)
            plsc.addupdate_scatter(
                deg_v, [d16 >> 7, d16 & 127],
                cnt.astype(jnp.float32), mask=last)
            return 0
        lax.fori_loop(0, DCH // 16, _deg, 0, unroll=4)

    plsc.subcore_barrier()

    pltpu.sync_copy(deg_v, deg_sh.at[idxq.at[0]], add=True)

    plsc.subcore_barrier()

    pltpu.sync_copy(deg_sh, deg_v)

    def _newton(i, _):
        for q in range(8):
            d = deg_v[i, pl.ds(q * 16, 16)] + 1.0
            deg_v[i, pl.ds(q * 16, 16)] = _rsqrt_newton(d)
        return 0
    lax.fori_loop(0, DROW, _newton, 0, unroll=2)

    def _xs_chunk(jj, _):
        base = s * 625 + jnp.minimum(jj * 32, 593)
        for q in range(2):
            iq1[0, pl.ds(q * 16, 16)] = base + q * 16 + iota16
        pltpu.async_copy(x_hbm.at[iq1.at[0]], rows.at[0], sem).wait()

        def _scale_row(r, _2):
            g = base + r
            w = plsc.load_gather(
                deg_v, [jnp.full((16,), g >> 7, jnp.int32),
                        jnp.full((16,), g & 127, jnp.int32)])
            for q in range(8):
                rows[0, r, pl.ds(q * 16, 16)] = (
                    rows[0, r, pl.ds(q * 16, 16)] * w)
            return 0
        lax.fori_loop(0, 32, _scale_row, 0, unroll=4)
        pltpu.async_copy(rows.at[0], xs_out.at[c].at[iq1.at[0]], sem).wait()
        return 0
    lax.fori_loop(0, 20, _xs_chunk, 0)

    cnt16 = zi16
    for j, csz in enumerate(ECH_SIZES):
        ebase = wid * PT + j * ECH
        pltpu.sync_copy(src_hbm.at[pl.ds(ebase, csz)],
                        ebuf_s.at[pl.ds(0, csz)])
        pltpu.sync_copy(dst_hbm.at[pl.ds(ebase, csz)],
                        ebuf_d.at[pl.ds(0, csz)])

        def _compact(i, cnt16):
            s16 = ebuf_s[pl.ds(i * 16, 16)]
            d16 = ebuf_d[pl.ds(i * 16, 16)]
            p16 = plsc.load_gather(pos_v, [d16])
            m = p16 < K
            pre = jnp.cumsum(m.astype(jnp.int32))
            offs = jnp.minimum(cnt16 + pre - 1, CAP - 1)
            plsc.store_scatter(ec, [offs >> 6, offs & 63],
                               (p16 << 14) | s16, mask=m)
            return cnt16 + pre[15]
        cnt16 = lax.fori_loop(0, csz // 16, _compact, cnt16, unroll=4)
    cnt = jnp.minimum(cnt16[0], CAP)
    nch = (cnt + 31) >> 5

    def _pad(t, _):
        idx = t * 16 + iota16
        m = idx >= cnt
        plsc.store_scatter(ec, [idx >> 6, idx & 63],
                           jnp.full((16,), 1023 << 14, jnp.int32), mask=m)
        return 0
    lax.fori_loop(cnt >> 4, nch * 2, _pad, 0)

    plsc.subcore_barrier()

    xs_c = xs_out.at[c]

    def _dec(g, b):
        for q in range(2):
            v = ec[g >> 1, pl.ds((g & 1) * 32 + q * 16, 16)]
            iq2[b, pl.ds(q * 16, 16)] = v & 16383
            iq3[b, pl.ds(q * 16, 16)] = v >> 14

    def _gather_start(b):
        pltpu.async_copy(xs_c.at[iq2.at[b]], rows.at[b], gsem.at[b])

    def _gather_wait(b):
        pltpu.make_async_copy(xs_c.at[iq2.at[b]], rows.at[b],
                              gsem.at[b]).wait()

    def _scat_start(b):
        pltpu.async_copy(rows.at[b], acc_sh.at[iq3.at[b]], ssem.at[b],
                         add=True)

    def _scat_wait(b):
        pltpu.make_async_copy(rows.at[b], acc_sh.at[iq3.at[b]],
                              ssem.at[b]).wait()

    @pl.when(nch > 0)
    def _():
        _dec(0, 0)
        _gather_start(0)

    def _chunk(g, _):
        b = g & 1
        nb = 1 - b

        @pl.when(g + 1 < nch)
        def _():
            @pl.when(g >= 1)
            def _():
                _scat_wait(nb)
            _dec(g + 1, nb)
            _gather_start(nb)
        _gather_wait(b)
        _scat_start(b)
        return 0
    lax.fori_loop(0, nch, _chunk, 0)

    @pl.when(nch >= 2)
    def _():
        _scat_wait(nch & 1)
    @pl.when(nch >= 1)
    def _():
        _scat_wait((nch - 1) & 1)

    plsc.subcore_barrier()

    for h in range(2):
        for q in range(2):
            iq1[h, pl.ds(q * 16, 16)] = s * 64 + h * 32 + q * 16 + iota16
        pltpu.sync_copy(acc_sh.at[pl.ds(s * 64 + h * 32, 32)], rows.at[h])
        pltpu.async_copy(rows.at[h], acc_out.at[c].at[iq1.at[h]],
                         sem).wait()

    @pl.when((c == 0) & (s == 0))
    def _():
        pltpu.sync_copy(sel_hbm, ebuf_s.at[pl.ds(0, KP)])

        def _dsel(k, _):
            i16 = ebuf_s[pl.ds(k * 16, 16)]
            v16 = plsc.load_gather(deg_v, [i16 >> 7, i16 & 127])
            ebuf_d[pl.ds(k * 16, 16)] = plsc.bitcast(v16, jnp.int32)
            return 0
        lax.fori_loop(0, KP // 16, _dsel, 0)
        pltpu.sync_copy(ebuf_d.at[pl.ds(0, KP)], dsel_out)


_sc_kernel = functools.partial(
    pl.kernel,
    out_type=(
        jax.ShapeDtypeStruct((2, KP, D), jnp.float32),
        jax.ShapeDtypeStruct((KP,), jnp.int32),
        jax.ShapeDtypeStruct((2, N, D), jnp.float32),
    ),
    mesh=plsc.VectorSubcoreMesh(core_axis_name="c", subcore_axis_name="s"),
    compiler_params=pltpu.CompilerParams(needs_layout_passes=False),
    scratch_types=(
        pltpu.VMEM((NP,), jnp.int32),
        pltpu.VMEM((DROW, 128), jnp.float32),
        pltpu.VMEM((ECH,), jnp.int32),
        pltpu.VMEM((DCH,), jnp.int32),
        pltpu.VMEM((CROWS, 64), jnp.int32),
        pltpu.VMEM((2, 32, D), jnp.float32),
        pltpu.VMEM((1, 80), jnp.int32),
        pltpu.VMEM((2, 32), jnp.int32),
        pltpu.VMEM((2, 32), jnp.int32),
        pltpu.VMEM((2, 32), jnp.int32),
        pltpu.VMEM_SHARED((DROW, 128), jnp.float32),
        pltpu.VMEM_SHARED((KP, D), jnp.float32),
        pltpu.SemaphoreType.DMA,
        pltpu.SemaphoreType.DMA((2,)),
        pltpu.SemaphoreType.DMA((2,)),
    ),
)(_sc_body)


def _tc_body(acc_ref, dsel_ref, w1_ref, b1_ref, w2_ref, b2_ref, out_ref):
    a = (acc_ref[0] + acc_ref[1]) * dsel_ref[...]
    z = jnp.dot(a, w1_ref[...], precision=jax.lax.Precision.HIGHEST)
    z = jnp.maximum(z + b1_ref[...], 0.0)
    o = jnp.dot(z, w2_ref[...], precision=jax.lax.Precision.HIGHEST)
    o = o + b2_ref[...]
    m = jnp.max(o, axis=1, keepdims=True)
    lse = jnp.log(jnp.sum(jnp.exp(o - m), axis=1, keepdims=True)) + m
    out_ref[...] = (o - lse)[:K]


def kernel(x, edge_index, W1, b1, W2, b2):
    sel, pos1d, sel_pad = _consts()
    src_ext = jnp.concatenate(
        [edge_index[0], sel, jnp.zeros((24,), jnp.int32)])
    dst_ext = jnp.concatenate(
        [edge_index[1], sel, jnp.full((24,), N, jnp.int32)])
    acc, dsel_bits, _ = _sc_kernel(
        src_ext, dst_ext, edge_index[1], pos1d, sel_pad, x)
    dsel = lax.bitcast_convert_type(dsel_bits, jnp.float32)
    out = pl.pallas_call(
        _tc_body,
        out_shape=jax.ShapeDtypeStruct((K, D_OUT), jnp.float32),
    )(acc, dsel.reshape(KP, 1), W1, b1.reshape(1, D), W2,
      b2.reshape(1, D_OUT))
    return out
